# baseline (speedup 1.0000x reference)
import functools

import jax
import jax.numpy as jnp
from jax import lax
from jax.experimental import pallas as pl
from jax.experimental.pallas import tpu as pltpu
from jax.experimental.pallas import tpu_sc as plsc

VOCAB = 100000
EMBED = 64
N_ROWS = 16384
N_COLS = 50

NUM_CORES = 2
NUM_SUBCORES = 16
NW = NUM_CORES * NUM_SUBCORES
R_PER_W = N_ROWS // NW
BT_PER_W = R_PER_W // 128
NUNITS = N_COLS * BT_PER_W


def _make_gather():
    mesh = plsc.VectorSubcoreMesh(core_axis_name="c", subcore_axis_name="s")

    @functools.partial(
        pl.kernel,
        mesh=mesh,
        out_type=jax.ShapeDtypeStruct((N_COLS, 8, 128, 8, 128), jnp.float32),
        scratch_types=[
            pltpu.VMEM((R_PER_W, N_COLS), jnp.int32),
            pltpu.VMEM((N_COLS, BT_PER_W, 128), jnp.int32),
            pltpu.VMEM((128, EMBED), jnp.float32),
            pltpu.VMEM((128, EMBED), jnp.float32),
            pltpu.VMEM((8, 8, 128), jnp.float32),
            pltpu.VMEM((8, 8, 128), jnp.float32),
            pltpu.SemaphoreType.DMA,
            pltpu.SemaphoreType.DMA,
            pltpu.SemaphoreType.DMA,
            pltpu.SemaphoreType.DMA,
        ],
        compiler_params=pltpu.CompilerParams(
            use_tc_tiling_on_sc=False, needs_layout_passes=False),
    )
    def gather_kernel(idx_hbm, table_hbm, out_hbm, idx_v, idx_t,
                      rows0, rows1, tr0, tr1, g0, g1, s0, s1):
        wid = lax.axis_index("s") * NUM_CORES + lax.axis_index("c")
        base_row = wid * R_PER_W
        pltpu.sync_copy(idx_hbm.at[pl.ds(base_row, R_PER_W)], idx_v)

        iota16 = lax.iota(jnp.int32, 16)

        def build_col(s, carry):
            sv = lax.broadcast(s, (16,))
            for bloc in range(BT_PER_W):
                for bc in range(8):
                    rowv = iota16 + (bloc * 128 + bc * 16)
                    v = plsc.load_gather(idx_v, [rowv, sv])
                    idx_t[s, bloc, pl.ds(bc * 16, 16)] = v
            return carry

        lax.fori_loop(0, N_COLS, build_col, 0)

        def fg(u, rows, gsem):
            s = u // BT_PER_W
            bloc = lax.rem(u, BT_PER_W)
            pltpu.async_copy(table_hbm.at[idx_t.at[s, bloc]], rows, gsem)

        def dg(rows, gsem):
            pltpu.make_async_copy(
                table_hbm.at[pl.ds(0, 128)], rows, gsem).wait()

        def transpose(rows, tr):
            def per_et(et, carry):
                for ei in range(8):
                    ev = lax.broadcast(et * 8 + ei, (16,))
                    for bc in range(8):
                        bv = iota16 + bc * 16
                        v = plsc.load_gather(rows, [bv, ev])
                        tr[et, ei, pl.ds(bc * 16, 16)] = v
                return carry

            lax.fori_loop(0, 8, per_et, 0)

        def fs(u, tr, ssem):
            s = u // BT_PER_W
            btg = wid * BT_PER_W + lax.rem(u, BT_PER_W)
            for et in range(8):
                pltpu.async_copy(tr.at[et], out_hbm.at[s, et, btg], ssem)

        def ds(tr, ssem):
            for et in range(8):
                pltpu.make_async_copy(
                    tr.at[et], out_hbm.at[0, 0, 0], ssem).wait()

        def unit(u, rows, tr, gsem, ssem, first):
            dg(rows, gsem)
            if not first:
                ds(tr, ssem)
            transpose(rows, tr)

            @pl.when(u + 2 < NUNITS)
            def _():
                fg(u + 2, rows, gsem)

            fs(u, tr, ssem)

        fg(0, rows0, g0)
        fg(1, rows1, g1)
        unit(jnp.int32(0), rows0, tr0, g0, s0, first=True)
        unit(jnp.int32(1), rows1, tr1, g1, s1, first=True)

        def body(p, carry):
            unit(2 * p, rows0, tr0, g0, s0, first=False)
            unit(2 * p + 1, rows1, tr1, g1, s1, first=False)
            return carry

        lax.fori_loop(1, NUNITS // 2, body, 0)

        ds(tr0, s0)
        ds(tr1, s1)

    return gather_kernel


_gather = _make_gather()


def kernel(indices, table):
    p = _gather(indices.astype(jnp.int32), table)
    return jnp.transpose(p, (2, 4, 0, 1, 3)).reshape(N_ROWS, N_COLS, EMBED)
